# baseline (speedup 1.0000x reference)
import functools

import jax
import jax.numpy as jnp
from jax import lax
from jax.experimental import pallas as pl
from jax.experimental.pallas import tpu as pltpu
from jax.experimental.pallas import tpu_sc as plsc

NUM_PUZZLES = 1000000
EMB_DIM = 64
BATCH = 16384
_HALF = NUM_PUZZLES // 2

_info = plsc.get_sparse_core_info()
_NC, _NS, _NL = _info.num_cores, _info.num_subcores, _info.num_lanes
_NW = _NC * _NS
_B_PER_W = BATCH // _NW


def _make_gather():
  mesh = plsc.VectorSubcoreMesh(core_axis_name="c", subcore_axis_name="s")

  @functools.partial(
      pl.kernel,
      mesh=mesh,
      compiler_params=pltpu.CompilerParams(
          use_tc_tiling_on_sc=False,
          needs_layout_passes=False,
      ),
      out_type=jax.ShapeDtypeStruct((BATCH, EMB_DIM), jnp.float32),
      scratch_types=[
          pltpu.VMEM((_B_PER_W,), jnp.int32),
          pltpu.VMEM((_B_PER_W,), jnp.int32),
          pltpu.VMEM((_B_PER_W,), jnp.int32),
          pltpu.VMEM((2, _B_PER_W, EMB_DIM), jnp.float32),
          pltpu.VMEM((_B_PER_W, EMB_DIM), jnp.float32),
          pltpu.SemaphoreType.DMA,
      ],
  )
  def gather_kernel(
      idx_hbm, taba_hbm, tabb_hbm, out_hbm,
      idx_v, ida, idb, rows2, stage, sem,
  ):
    wid = lax.axis_index("s") * _NC + lax.axis_index("c")
    base = wid * _B_PER_W
    pltpu.sync_copy(idx_hbm.at[pl.ds(base, _B_PER_W)], idx_v)

    def split_body(k, carry):
      sel = pl.ds(k * _NL, _NL)
      v = idx_v[sel]
      in_b = v >= _HALF
      ida[sel] = jnp.where(in_b, 0, v)
      idb[sel] = jnp.where(in_b, v - _HALF, 0)
      return carry

    lax.fori_loop(0, _B_PER_W // _NL, split_body, 0)

    copy_a = pltpu.make_async_copy(taba_hbm.at[ida], rows2.at[0], sem)
    copy_a.start()
    copy_b = pltpu.make_async_copy(tabb_hbm.at[idb], rows2.at[1], sem)
    copy_b.start()
    copy_a.wait()
    copy_b.wait()

    lanes = lax.iota(jnp.int32, _NL)

    def sel_body(g, carry):
      v = idx_v[pl.ds(g * _NL, _NL)]
      s16 = jnp.where(v >= _HALF, 1, 0)
      i16 = lanes + g * _NL
      for col in range(EMB_DIM):
        c16 = jnp.full((_NL,), col, jnp.int32)
        val = plsc.load_gather(rows2, [s16, i16, c16])
        plsc.store_scatter(stage, [i16, c16], val)
      return carry

    lax.fori_loop(0, _B_PER_W // _NL, sel_body, 0)
    pltpu.sync_copy(stage, out_hbm.at[pl.ds(base, _B_PER_W)])

  return gather_kernel


_gather = _make_gather()


@jax.jit
def kernel(puzzle_ids, embeddings):
  if puzzle_ids.ndim > 1:
    puzzle_ids = jnp.squeeze(puzzle_ids, axis=-1)
  return _gather(
      puzzle_ids.astype(jnp.int32),
      embeddings[:_HALF],
      embeddings[_HALF:],
  )


# speedup vs baseline: 1.9408x; 1.9408x over previous
import functools

import jax
import jax.numpy as jnp
from jax import lax
from jax.experimental import pallas as pl
from jax.experimental.pallas import tpu as pltpu
from jax.experimental.pallas import tpu_sc as plsc

NUM_PUZZLES = 1000000
EMB_DIM = 64
BATCH = 16384

_info = plsc.get_sparse_core_info()
_NS = _info.num_subcores
_B_PER_W = BATCH // _NS


def _make_gather():
  mesh = plsc.VectorSubcoreMesh(
      core_axis_name="c", subcore_axis_name="s", num_cores=1
  )

  @functools.partial(
      pl.kernel,
      mesh=mesh,
      compiler_params=pltpu.CompilerParams(use_tc_tiling_on_sc=False),
      out_type=jax.ShapeDtypeStruct((BATCH, EMB_DIM), jnp.float32),
      scratch_types=[
          pltpu.VMEM((_B_PER_W,), jnp.int32),
          pltpu.VMEM((_B_PER_W, EMB_DIM), jnp.float32),
          pltpu.SemaphoreType.DMA,
      ],
  )
  def gather_kernel(idx_hbm, table_hbm, out_hbm, idx_v, rows_v, sem):
    wid = lax.axis_index("s")
    base = wid * _B_PER_W
    pltpu.sync_copy(idx_hbm.at[pl.ds(base, _B_PER_W)], idx_v)
    pltpu.async_copy(table_hbm.at[idx_v], rows_v, sem).wait()
    pltpu.sync_copy(rows_v, out_hbm.at[pl.ds(base, _B_PER_W)])

  return gather_kernel


_gather = _make_gather()


@jax.jit
def kernel(puzzle_ids, embeddings):
  if puzzle_ids.ndim > 1:
    puzzle_ids = jnp.squeeze(puzzle_ids, axis=-1)
  return _gather(puzzle_ids.astype(jnp.int32), embeddings)


# speedup vs baseline: 3.3456x; 1.7238x over previous
import functools

import jax
import jax.numpy as jnp
from jax import lax
from jax.experimental import pallas as pl
from jax.experimental.pallas import tpu as pltpu
from jax.experimental.pallas import tpu_sc as plsc

NUM_PUZZLES = 1000000
EMB_DIM = 64
BATCH = 16384

_info = plsc.get_sparse_core_info()
_NC, _NS, _NL = _info.num_cores, _info.num_subcores, _info.num_lanes
_NW = _NC * _NS
_B_PER_W = BATCH // _NW
_N_CHUNKS = _B_PER_W // _NL


def _make_gather():
  mesh = plsc.VectorSubcoreMesh(core_axis_name="c", subcore_axis_name="s")

  @functools.partial(
      pl.kernel,
      mesh=mesh,
      compiler_params=pltpu.CompilerParams(needs_layout_passes=False),
      out_type=jax.ShapeDtypeStruct((BATCH, EMB_DIM), jnp.float32),
      scratch_types=[
          pltpu.VMEM((_B_PER_W,), jnp.int32),
          pltpu.VMEM((_B_PER_W, EMB_DIM), jnp.float32),
          pltpu.SemaphoreType.DMA,
      ],
  )
  def gather_kernel(idx_hbm, table_hbm, out_hbm, idx_v, rows_v, sem):
    wid = lax.axis_index("c") * _NS + lax.axis_index("s")
    base = wid * _B_PER_W
    pltpu.sync_copy(idx_hbm.at[pl.ds(base, _B_PER_W)], idx_v)
    lanes = lax.iota(jnp.int32, _NL)

    def body(chunk, carry):
      vec = idx_v[pl.ds(chunk * _NL, _NL)]
      for j in range(_NL):
        row = jnp.sum(jnp.where(lanes == j, vec, 0))
        pltpu.async_copy(
            table_hbm.at[pl.ds(row, 1)],
            rows_v.at[pl.ds(chunk * _NL + j, 1)],
            sem,
        )
      return carry

    lax.fori_loop(0, _N_CHUNKS, body, 0)
    pltpu.make_async_copy(table_hbm.at[pl.ds(0, _B_PER_W)], rows_v, sem).wait()
    pltpu.sync_copy(rows_v, out_hbm.at[pl.ds(base, _B_PER_W)])

  return gather_kernel


_gather = _make_gather()


@jax.jit
def kernel(puzzle_ids, embeddings):
  if puzzle_ids.ndim > 1:
    puzzle_ids = jnp.squeeze(puzzle_ids, axis=-1)
  return _gather(puzzle_ids.astype(jnp.int32), embeddings)


# speedup vs baseline: 3.3493x; 1.0011x over previous
import functools

import jax
import jax.numpy as jnp
from jax import lax
from jax.experimental import pallas as pl
from jax.experimental.pallas import tpu as pltpu
from jax.experimental.pallas import tpu_sc as plsc

NUM_PUZZLES = 1000000
EMB_DIM = 64
BATCH = 16384

_info = plsc.get_sparse_core_info()
_NC, _NS, _NL = _info.num_cores, _info.num_subcores, _info.num_lanes
_NW = _NC * _NS
_B_PER_W = BATCH // _NW
_N_CHUNKS = _B_PER_W // _NL


def _make_gather():
  mesh = plsc.VectorSubcoreMesh(core_axis_name="c", subcore_axis_name="s")

  @functools.partial(
      pl.kernel,
      mesh=mesh,
      compiler_params=pltpu.CompilerParams(needs_layout_passes=False),
      out_type=jax.ShapeDtypeStruct((BATCH, EMB_DIM), jnp.float32),
      scratch_types=[
          pltpu.VMEM((_B_PER_W,), jnp.int32),
          pltpu.VMEM((_B_PER_W, EMB_DIM), jnp.float32),
          pltpu.SemaphoreType.DMA,
          pltpu.SemaphoreType.DMA,
          pltpu.SemaphoreType.DMA,
          pltpu.SemaphoreType.DMA,
      ],
  )
  def gather_kernel(
      idx_hbm, table_hbm, out_hbm, idx_v, rows_v, sem, sem1, sem2, sem3
  ):
    sems = (sem, sem1, sem2, sem3)
    wid = lax.axis_index("c") * _NS + lax.axis_index("s")
    base = wid * _B_PER_W
    pltpu.sync_copy(idx_hbm.at[pl.ds(base, _B_PER_W)], idx_v)
    lanes = lax.iota(jnp.int32, _NL)

    def body(chunk, carry):
      vec = idx_v[pl.ds(chunk * _NL, _NL)]
      for j in range(_NL):
        row = jnp.sum(jnp.where(lanes == j, vec, 0))
        pltpu.async_copy(
            table_hbm.at[pl.ds(row, 1)],
            rows_v.at[pl.ds(chunk * _NL + j, 1)],
            sems[j % 4],
        )
      return carry

    lax.fori_loop(0, _N_CHUNKS, body, 0)
    for q in range(4):
      pltpu.make_async_copy(
          table_hbm.at[pl.ds(0, _B_PER_W // 4)],
          rows_v.at[pl.ds(0, _B_PER_W // 4)],
          sems[q],
      ).wait()
    pltpu.sync_copy(rows_v, out_hbm.at[pl.ds(base, _B_PER_W)])

  return gather_kernel


_gather = _make_gather()


@jax.jit
def kernel(puzzle_ids, embeddings):
  if puzzle_ids.ndim > 1:
    puzzle_ids = jnp.squeeze(puzzle_ids, axis=-1)
  return _gather(puzzle_ids.astype(jnp.int32), embeddings)
